# baseline (speedup 1.0000x reference)
import functools

import jax
import jax.numpy as jnp
from jax import lax
from jax.experimental import pallas as pl
from jax.experimental.pallas import tpu as pltpu
from jax.experimental.pallas import tpu_sc as plsc

DIM = 32
EMB_RANGE = 14.0 / 500.0
PI = 3.141592653589793
_PHASE_DIV = EMB_RANGE / PI

_LANES = 16
_CH = 64
_PITCH = _LANES + 1
_TBLK = 4096


def _pack_tables(re_t, im_t):
    n = re_t.shape[1]
    grid = (n + _TBLK - 1) // _TBLK

    def body(a_ref, b_ref, oa_ref, ob_ref):
        ta = a_ref[...].T.reshape(_TBLK // 4, 4, DIM)
        tb = b_ref[...].T.reshape(_TBLK // 4, 4, DIM)
        oa_ref[...] = jnp.concatenate(
            [ta[:, jj, :] for jj in range(4)], axis=1)
        ob_ref[...] = jnp.concatenate(
            [tb[:, jj, :] for jj in range(4)], axis=1)

    spec_in = pl.BlockSpec((DIM, _TBLK), lambda g: (0, g))
    spec_out = pl.BlockSpec((_TBLK // 4, 128), lambda g: (g, 0))
    return pl.pallas_call(
        body,
        grid=(grid,),
        in_specs=[spec_in, spec_in],
        out_specs=[spec_out, spec_out],
        out_shape=[jax.ShapeDtypeStruct((n // 4, 128), jnp.float32)] * 2,
    )(re_t, im_t)


def _rel_tables(rel_w128):
    def body(rel_ref, rr_ref, ir_ref):
        ph = rel_ref[...] / jnp.float32(_PHASE_DIV)
        rr_ref[...] = jnp.cos(ph)
        ir_ref[...] = jnp.sin(ph)

    return pl.pallas_call(
        body,
        out_shape=[jax.ShapeDtypeStruct(rel_w128.shape, jnp.float32)] * 2,
    )(rel_w128)


def _vsqrt(x):
    x = jnp.maximum(x, jnp.float32(1e-30))
    i = lax.bitcast_convert_type(x, jnp.int32)
    i = jnp.int32(0x5F3759DF) - lax.shift_right_arithmetic(i, jnp.int32(1))
    y = lax.bitcast_convert_type(i, jnp.float32)
    half_x = jnp.float32(0.5) * x
    for _ in range(2):
        y = y * (jnp.float32(1.5) - half_x * y * y)
    return x * y


def _sc_score(h, t, r, rw4, iw4, rr_tab, ir_tab):
    rows = h.shape[0]
    mesh = plsc.VectorSubcoreMesh(core_axis_name="c", subcore_axis_name="s")
    nc, ns = mesh.num_cores, mesh.num_subcores
    nw = nc * ns
    bpw = rows // nw
    nch = bpw // _CH
    assert bpw * nw == rows and nch * _CH == bpw and nch % 2 == 0

    @functools.partial(
        pl.kernel,
        out_type=jax.ShapeDtypeStruct((rows,), jnp.float32),
        mesh=mesh,
        scratch_types=[
            pltpu.VMEM((bpw,), jnp.int32),
            pltpu.VMEM((bpw,), jnp.int32),
            pltpu.VMEM((bpw,), jnp.int32),
            pltpu.VMEM((bpw,), jnp.int32),
            pltpu.VMEM((bpw,), jnp.int32),
            pltpu.VMEM((bpw,), jnp.int32),
            pltpu.VMEM((2, 6 * _CH, 128), jnp.float32),
            pltpu.VMEM((_LANES * _PITCH,), jnp.float32),
            pltpu.VMEM((_CH,), jnp.float32),
            pltpu.SemaphoreType.DMA,
            pltpu.SemaphoreType.DMA,
        ],
        compiler_params=pltpu.CompilerParams(needs_layout_passes=False),
    )
    def k(h_hbm, t_hbm, r_hbm, rew_hbm, imw_hbm, rrt_hbm, irt_hbm, out_hbm,
          hidx, tidx, ridx, hp, tp, rp, buf, sc, outv, sem0, sem1):
        cid = lax.axis_index("c")
        sid = lax.axis_index("s")
        wid = sid * nc + cid
        base = wid * bpw
        pltpu.sync_copy(h_hbm.at[pl.ds(base, bpw)], hidx)
        pltpu.sync_copy(t_hbm.at[pl.ds(base, bpw)], tidx)
        pltpu.sync_copy(r_hbm.at[pl.ds(base, bpw)], ridx)

        def pack_body(j, carry):
            o = j * _LANES
            hp[pl.ds(o, _LANES)] = lax.shift_right_logical(
                hidx[pl.ds(o, _LANES)], 2)
            tp[pl.ds(o, _LANES)] = lax.shift_right_logical(
                tidx[pl.ds(o, _LANES)], 2)
            rp[pl.ds(o, _LANES)] = lax.shift_right_logical(
                ridx[pl.ds(o, _LANES)], 2)
            return carry

        lax.fori_loop(0, bpw // _LANES, pack_body, 0)

        row_iota = lax.iota(jnp.int32, _LANES)
        col_iota = row_iota * _PITCH
        sems = (sem0, sem1)

        def issue(cc, slot):
            off = cc * _CH
            bslot = buf.at[slot]
            sem = sems[slot]
            pltpu.async_copy(rew_hbm.at[hp.at[pl.ds(off, _CH)]],
                             bslot.at[pl.ds(0, _CH)], sem)
            pltpu.async_copy(rew_hbm.at[tp.at[pl.ds(off, _CH)]],
                             bslot.at[pl.ds(_CH, _CH)], sem)
            pltpu.async_copy(imw_hbm.at[hp.at[pl.ds(off, _CH)]],
                             bslot.at[pl.ds(2 * _CH, _CH)], sem)
            pltpu.async_copy(imw_hbm.at[tp.at[pl.ds(off, _CH)]],
                             bslot.at[pl.ds(3 * _CH, _CH)], sem)
            pltpu.async_copy(rrt_hbm.at[rp.at[pl.ds(off, _CH)]],
                             bslot.at[pl.ds(4 * _CH, _CH)], sem)
            pltpu.async_copy(irt_hbm.at[rp.at[pl.ds(off, _CH)]],
                             bslot.at[pl.ds(5 * _CH, _CH)], sem)

        def drain(slot):
            pltpu.make_async_copy(
                rew_hbm.at[pl.ds(0, 6 * _CH)], buf.at[slot], sems[slot]).wait()

        def compute(cc, slot):
            off = cc * _CH
            bslot = buf.at[slot]

            def row_body(g, inner):
                goff = off + g * _LANES
                hqv = lax.shift_left(hidx[pl.ds(goff, _LANES)] & 3, 5)
                tqv = lax.shift_left(tidx[pl.ds(goff, _LANES)] & 3, 5)
                rqv = lax.shift_left(ridx[pl.ds(goff, _LANES)] & 3, 5)
                for u in range(_LANES):
                    rr = g * _LANES + u
                    hq = hqv[u]
                    tq = tqv[u]
                    rq = rqv[u]
                    sv = None
                    for o in (0, _LANES):
                        rh = bslot[rr, pl.ds(hq + o, _LANES)]
                        rt = bslot[_CH + rr, pl.ds(tq + o, _LANES)]
                        ih = bslot[2 * _CH + rr, pl.ds(hq + o, _LANES)]
                        it = bslot[3 * _CH + rr, pl.ds(tq + o, _LANES)]
                        rrel = bslot[4 * _CH + rr, pl.ds(rq + o, _LANES)]
                        irel = bslot[5 * _CH + rr, pl.ds(rq + o, _LANES)]
                        re = rh * rt + irel * it - rh
                        im = rrel * it - irel * rh - ih
                        s = _vsqrt(re * re + im * im)
                        sv = s if sv is None else sv + s
                    sc[pl.ds(u * _PITCH, _LANES)] = sv
                acc = None
                for i in range(_LANES):
                    col = plsc.load_gather(sc, [col_iota + i])
                    acc = col if acc is None else acc + col
                outv[pl.ds(g * _LANES, _LANES)] = jnp.float32(12.0) - acc
                return inner

            lax.fori_loop(0, _CH // _LANES, row_body, 0)
            pltpu.sync_copy(outv, out_hbm.at[pl.ds(base + off, _CH)])

        issue(0, 0)

        def pipe_body(i, carry):
            c0 = i * 2
            issue(c0 + 1, 1)
            drain(0)
            compute(c0, 0)

            @pl.when(c0 + 2 < nch)
            def _():
                issue(c0 + 2, 0)

            drain(1)
            compute(c0 + 1, 1)
            return carry

        lax.fori_loop(0, nch // 2, pipe_body, 0)

    return k(h, t, r, rw4, iw4, rr_tab, ir_tab)


def kernel(heads, tails, relations, negative_heads, negative_tails,
           negative_relations, re_ent_w, im_ent_w, rel_w):
    b = heads.shape[0]
    rw4, iw4 = _pack_tables(re_ent_w.T, im_ent_w.T)
    rr_tab, ir_tab = _rel_tables(rel_w.reshape(-1, 128))
    h = jnp.concatenate([heads, negative_heads]).astype(jnp.int32)
    t = jnp.concatenate([tails, negative_tails]).astype(jnp.int32)
    r = jnp.concatenate([relations, negative_relations]).astype(jnp.int32)
    out = _sc_score(h, t, r, rw4, iw4, rr_tab, ir_tab)
    return out[:b], out[b:]


# speedup vs baseline: 1.4021x; 1.4021x over previous
import functools

import jax
import jax.numpy as jnp
from jax import lax
from jax.experimental import pallas as pl
from jax.experimental.pallas import tpu as pltpu
from jax.experimental.pallas import tpu_sc as plsc

DIM = 32
EMB_RANGE = 14.0 / 500.0
PI = 3.141592653589793
_PHASE_DIV = EMB_RANGE / PI

_LANES = 16
_CH = 64
_PITCH = _LANES + 1
_SBAND = 1 << 18
_TBR = 2048


def _pack_tables(re_t, im_t):
    def body(a0, a1, a2, a3, b0, b1, b2, b3, oa_ref, ob_ref):
        for b, (ar, br) in enumerate(((a0, b0), (a1, b1), (a2, b2), (a3, b3))):
            oa_ref[:, pl.ds(DIM * b, DIM)] = ar[...].T
            ob_ref[:, pl.ds(DIM * b, DIM)] = br[...].T

    grid = _SBAND // _TBR
    n = re_t.shape[1]
    last_blk = (n + _TBR - 1) // _TBR - 1
    in_specs = []
    for _tbl in range(2):
        for b in range(4):
            in_specs.append(pl.BlockSpec(
                (DIM, _TBR),
                lambda g, b=b: (0, jnp.minimum(g + (_SBAND // _TBR) * b,
                                               last_blk))))
    spec_out = pl.BlockSpec((_TBR, 128), lambda g: (g, 0))
    return pl.pallas_call(
        body,
        grid=(grid,),
        in_specs=in_specs,
        out_specs=[spec_out, spec_out],
        out_shape=[jax.ShapeDtypeStruct((_SBAND, 128), jnp.float32)] * 2,
    )(re_t, re_t, re_t, re_t, im_t, im_t, im_t, im_t)


def _rel_tables(rel_w128):
    def body(rel_ref, rr_ref, ir_ref):
        ph = rel_ref[...] / jnp.float32(_PHASE_DIV)
        rr_ref[...] = jnp.cos(ph)
        ir_ref[...] = jnp.sin(ph)

    return pl.pallas_call(
        body,
        out_shape=[jax.ShapeDtypeStruct(rel_w128.shape, jnp.float32)] * 2,
    )(rel_w128)


def _vsqrt(x):
    x = jnp.maximum(x, jnp.float32(1e-30))
    i = lax.bitcast_convert_type(x, jnp.int32)
    i = jnp.int32(0x5F3759DF) - lax.shift_right_arithmetic(i, jnp.int32(1))
    y = lax.bitcast_convert_type(i, jnp.float32)
    half_x = jnp.float32(0.5) * x
    for _ in range(2):
        y = y * (jnp.float32(1.5) - half_x * y * y)
    return x * y


def _sc_score(h, t, r, rw4, iw4, rr_tab, ir_tab):
    rows = h.shape[0]
    mesh = plsc.VectorSubcoreMesh(core_axis_name="c", subcore_axis_name="s")
    nc, ns = mesh.num_cores, mesh.num_subcores
    nw = nc * ns
    bpw = rows // nw
    nch = bpw // _CH
    assert bpw * nw == rows and nch * _CH == bpw and nch % 2 == 0

    @functools.partial(
        pl.kernel,
        out_type=jax.ShapeDtypeStruct((rows,), jnp.float32),
        mesh=mesh,
        scratch_types=[
            pltpu.VMEM((bpw,), jnp.int32),
            pltpu.VMEM((bpw,), jnp.int32),
            pltpu.VMEM((bpw,), jnp.int32),
            pltpu.VMEM((bpw,), jnp.int32),
            pltpu.VMEM((bpw,), jnp.int32),
            pltpu.VMEM((bpw,), jnp.int32),
            pltpu.VMEM((2, 6 * _CH, 128), jnp.float32),
            pltpu.VMEM((_LANES * _PITCH,), jnp.float32),
            pltpu.VMEM((_CH,), jnp.float32),
            pltpu.SemaphoreType.DMA,
            pltpu.SemaphoreType.DMA,
        ],
        compiler_params=pltpu.CompilerParams(needs_layout_passes=False),
    )
    def k(h_hbm, t_hbm, r_hbm, rew_hbm, imw_hbm, rrt_hbm, irt_hbm, out_hbm,
          hidx, tidx, ridx, hp, tp, rp, buf, sc, outv, sem0, sem1):
        cid = lax.axis_index("c")
        sid = lax.axis_index("s")
        wid = sid * nc + cid
        base = wid * bpw
        pltpu.sync_copy(h_hbm.at[pl.ds(base, bpw)], hidx)
        pltpu.sync_copy(t_hbm.at[pl.ds(base, bpw)], tidx)
        pltpu.sync_copy(r_hbm.at[pl.ds(base, bpw)], ridx)

        def pack_body(j, carry):
            o = j * _LANES
            hp[pl.ds(o, _LANES)] = hidx[pl.ds(o, _LANES)] & (_SBAND - 1)
            tp[pl.ds(o, _LANES)] = tidx[pl.ds(o, _LANES)] & (_SBAND - 1)
            rp[pl.ds(o, _LANES)] = lax.shift_right_logical(
                ridx[pl.ds(o, _LANES)], 2)
            return carry

        lax.fori_loop(0, bpw // _LANES, pack_body, 0)

        row_iota = lax.iota(jnp.int32, _LANES)
        col_iota = row_iota * _PITCH
        sems = (sem0, sem1)

        def issue(cc, slot):
            off = cc * _CH
            bslot = buf.at[slot]
            sem = sems[slot]
            pltpu.async_copy(rew_hbm.at[hp.at[pl.ds(off, _CH)]],
                             bslot.at[pl.ds(0, _CH)], sem)
            pltpu.async_copy(rew_hbm.at[tp.at[pl.ds(off, _CH)]],
                             bslot.at[pl.ds(_CH, _CH)], sem)
            pltpu.async_copy(imw_hbm.at[hp.at[pl.ds(off, _CH)]],
                             bslot.at[pl.ds(2 * _CH, _CH)], sem)
            pltpu.async_copy(imw_hbm.at[tp.at[pl.ds(off, _CH)]],
                             bslot.at[pl.ds(3 * _CH, _CH)], sem)
            pltpu.async_copy(rrt_hbm.at[rp.at[pl.ds(off, _CH)]],
                             bslot.at[pl.ds(4 * _CH, _CH)], sem)
            pltpu.async_copy(irt_hbm.at[rp.at[pl.ds(off, _CH)]],
                             bslot.at[pl.ds(5 * _CH, _CH)], sem)

        def drain(slot):
            pltpu.make_async_copy(
                rew_hbm.at[pl.ds(0, 6 * _CH)], buf.at[slot], sems[slot]).wait()

        def compute(cc, slot):
            off = cc * _CH
            bslot = buf.at[slot]

            def row_body(g, inner):
                goff = off + g * _LANES
                hqv = lax.shift_left(
                    lax.shift_right_logical(hidx[pl.ds(goff, _LANES)], 18), 5)
                tqv = lax.shift_left(
                    lax.shift_right_logical(tidx[pl.ds(goff, _LANES)], 18), 5)
                rqv = lax.shift_left(ridx[pl.ds(goff, _LANES)] & 3, 5)
                for u in range(_LANES):
                    rr = g * _LANES + u
                    hq = hqv[u]
                    tq = tqv[u]
                    rq = rqv[u]
                    sv = None
                    for o in (0, _LANES):
                        rh = bslot[rr, pl.ds(hq + o, _LANES)]
                        rt = bslot[_CH + rr, pl.ds(tq + o, _LANES)]
                        ih = bslot[2 * _CH + rr, pl.ds(hq + o, _LANES)]
                        it = bslot[3 * _CH + rr, pl.ds(tq + o, _LANES)]
                        rrel = bslot[4 * _CH + rr, pl.ds(rq + o, _LANES)]
                        irel = bslot[5 * _CH + rr, pl.ds(rq + o, _LANES)]
                        re = rh * rt + irel * it - rh
                        im = rrel * it - irel * rh - ih
                        s = _vsqrt(re * re + im * im)
                        sv = s if sv is None else sv + s
                    sc[pl.ds(u * _PITCH, _LANES)] = sv
                acc = None
                for i in range(_LANES):
                    col = plsc.load_gather(sc, [col_iota + i])
                    acc = col if acc is None else acc + col
                outv[pl.ds(g * _LANES, _LANES)] = jnp.float32(12.0) - acc
                return inner

            lax.fori_loop(0, _CH // _LANES, row_body, 0)
            pltpu.sync_copy(outv, out_hbm.at[pl.ds(base + off, _CH)])

        issue(0, 0)

        def pipe_body(i, carry):
            c0 = i * 2
            issue(c0 + 1, 1)
            drain(0)
            compute(c0, 0)

            @pl.when(c0 + 2 < nch)
            def _():
                issue(c0 + 2, 0)

            drain(1)
            compute(c0 + 1, 1)
            return carry

        lax.fori_loop(0, nch // 2, pipe_body, 0)

    return k(h, t, r, rw4, iw4, rr_tab, ir_tab)


def kernel(heads, tails, relations, negative_heads, negative_tails,
           negative_relations, re_ent_w, im_ent_w, rel_w):
    b = heads.shape[0]
    rw4, iw4 = _pack_tables(re_ent_w.T, im_ent_w.T)
    rr_tab, ir_tab = _rel_tables(rel_w.reshape(-1, 128))
    h = jnp.concatenate([heads, negative_heads]).astype(jnp.int32)
    t = jnp.concatenate([tails, negative_tails]).astype(jnp.int32)
    r = jnp.concatenate([relations, negative_relations]).astype(jnp.int32)
    out = _sc_score(h, t, r, rw4, iw4, rr_tab, ir_tab)
    return out[:b], out[b:]


# speedup vs baseline: 2.9559x; 2.1083x over previous
import functools

import jax
import jax.numpy as jnp
from jax import lax
from jax.experimental import pallas as pl
from jax.experimental.pallas import tpu as pltpu
from jax.experimental.pallas import tpu_sc as plsc

DIM = 32
EMB_RANGE = 14.0 / 500.0
PI = 3.141592653589793
_PHASE_DIV = EMB_RANGE / PI

_LANES = 16
_CH = 64
_PITCH = _LANES + 1
_SBAND = 1 << 18
_TBR = 2048


def _pack_tables(re_t, im_t):
    def body(a0, a1, a2, a3, b0, b1, b2, b3, oa_ref, ob_ref):
        sa = jnp.concatenate([a0[...], a1[...], a2[...], a3[...]], axis=0)
        sb = jnp.concatenate([b0[...], b1[...], b2[...], b3[...]], axis=0)
        oa_ref[...] = sa.T
        ob_ref[...] = sb.T

    grid = _SBAND // _TBR
    n = re_t.shape[1]
    last_blk = (n + _TBR - 1) // _TBR - 1
    in_specs = []
    for _tbl in range(2):
        for b in range(4):
            in_specs.append(pl.BlockSpec(
                (DIM, _TBR),
                lambda g, b=b: (0, jnp.minimum(g + (_SBAND // _TBR) * b,
                                               last_blk))))
    spec_out = pl.BlockSpec((_TBR, 128), lambda g: (g, 0))
    return pl.pallas_call(
        body,
        grid=(grid,),
        in_specs=in_specs,
        out_specs=[spec_out, spec_out],
        out_shape=[jax.ShapeDtypeStruct((_SBAND, 128), jnp.float32)] * 2,
    )(re_t, re_t, re_t, re_t, im_t, im_t, im_t, im_t)


def _rel_tables(rel_w128):
    def body(rel_ref, rr_ref, ir_ref):
        ph = rel_ref[...] / jnp.float32(_PHASE_DIV)
        rr_ref[...] = jnp.cos(ph)
        ir_ref[...] = jnp.sin(ph)

    return pl.pallas_call(
        body,
        out_shape=[jax.ShapeDtypeStruct(rel_w128.shape, jnp.float32)] * 2,
    )(rel_w128)


def _vsqrt(x):
    x = jnp.maximum(x, jnp.float32(1e-30))
    i = lax.bitcast_convert_type(x, jnp.int32)
    i = jnp.int32(0x5F3759DF) - lax.shift_right_arithmetic(i, jnp.int32(1))
    y = lax.bitcast_convert_type(i, jnp.float32)
    half_x = jnp.float32(0.5) * x
    for _ in range(2):
        y = y * (jnp.float32(1.5) - half_x * y * y)
    return x * y


def _sc_score(h, t, r, rw4, iw4, rr_tab, ir_tab):
    rows = h.shape[0]
    mesh = plsc.VectorSubcoreMesh(core_axis_name="c", subcore_axis_name="s")
    nc, ns = mesh.num_cores, mesh.num_subcores
    nw = nc * ns
    bpw = rows // nw
    nch = bpw // _CH
    assert bpw * nw == rows and nch * _CH == bpw and nch % 2 == 0

    @functools.partial(
        pl.kernel,
        out_type=jax.ShapeDtypeStruct((rows,), jnp.float32),
        mesh=mesh,
        scratch_types=[
            pltpu.VMEM((bpw,), jnp.int32),
            pltpu.VMEM((bpw,), jnp.int32),
            pltpu.VMEM((bpw,), jnp.int32),
            pltpu.VMEM((bpw,), jnp.int32),
            pltpu.VMEM((bpw,), jnp.int32),
            pltpu.VMEM((bpw,), jnp.int32),
            pltpu.VMEM((2, 6 * _CH, 128), jnp.float32),
            pltpu.VMEM((_LANES * _PITCH,), jnp.float32),
            pltpu.VMEM((_CH,), jnp.float32),
            pltpu.SemaphoreType.DMA,
            pltpu.SemaphoreType.DMA,
        ],
        compiler_params=pltpu.CompilerParams(needs_layout_passes=False),
    )
    def k(h_hbm, t_hbm, r_hbm, rew_hbm, imw_hbm, rrt_hbm, irt_hbm, out_hbm,
          hidx, tidx, ridx, hp, tp, rp, buf, sc, outv, sem0, sem1):
        cid = lax.axis_index("c")
        sid = lax.axis_index("s")
        wid = sid * nc + cid
        base = wid * bpw
        pltpu.sync_copy(h_hbm.at[pl.ds(base, bpw)], hidx)
        pltpu.sync_copy(t_hbm.at[pl.ds(base, bpw)], tidx)
        pltpu.sync_copy(r_hbm.at[pl.ds(base, bpw)], ridx)

        def pack_body(j, carry):
            o = j * _LANES
            hp[pl.ds(o, _LANES)] = hidx[pl.ds(o, _LANES)] & (_SBAND - 1)
            tp[pl.ds(o, _LANES)] = tidx[pl.ds(o, _LANES)] & (_SBAND - 1)
            rp[pl.ds(o, _LANES)] = lax.shift_right_logical(
                ridx[pl.ds(o, _LANES)], 2)
            return carry

        lax.fori_loop(0, bpw // _LANES, pack_body, 0)

        row_iota = lax.iota(jnp.int32, _LANES)
        col_iota = row_iota * _PITCH
        sems = (sem0, sem1)

        def issue(cc, slot):
            off = cc * _CH
            bslot = buf.at[slot]
            sem = sems[slot]
            pltpu.async_copy(rew_hbm.at[hp.at[pl.ds(off, _CH)]],
                             bslot.at[pl.ds(0, _CH)], sem)
            pltpu.async_copy(rew_hbm.at[tp.at[pl.ds(off, _CH)]],
                             bslot.at[pl.ds(_CH, _CH)], sem)
            pltpu.async_copy(imw_hbm.at[hp.at[pl.ds(off, _CH)]],
                             bslot.at[pl.ds(2 * _CH, _CH)], sem)
            pltpu.async_copy(imw_hbm.at[tp.at[pl.ds(off, _CH)]],
                             bslot.at[pl.ds(3 * _CH, _CH)], sem)
            pltpu.async_copy(rrt_hbm.at[rp.at[pl.ds(off, _CH)]],
                             bslot.at[pl.ds(4 * _CH, _CH)], sem)
            pltpu.async_copy(irt_hbm.at[rp.at[pl.ds(off, _CH)]],
                             bslot.at[pl.ds(5 * _CH, _CH)], sem)

        def drain(slot):
            pltpu.make_async_copy(
                rew_hbm.at[pl.ds(0, 6 * _CH)], buf.at[slot], sems[slot]).wait()

        def compute(cc, slot):
            off = cc * _CH
            bslot = buf.at[slot]

            def row_body(g, inner):
                goff = off + g * _LANES
                hqv = lax.shift_left(
                    lax.shift_right_logical(hidx[pl.ds(goff, _LANES)], 18), 5)
                tqv = lax.shift_left(
                    lax.shift_right_logical(tidx[pl.ds(goff, _LANES)], 18), 5)
                rqv = lax.shift_left(ridx[pl.ds(goff, _LANES)] & 3, 5)
                for u in range(_LANES):
                    rr = g * _LANES + u
                    hq = hqv[u]
                    tq = tqv[u]
                    rq = rqv[u]
                    sv = None
                    for o in (0, _LANES):
                        rh = bslot[rr, pl.ds(hq + o, _LANES)]
                        rt = bslot[_CH + rr, pl.ds(tq + o, _LANES)]
                        ih = bslot[2 * _CH + rr, pl.ds(hq + o, _LANES)]
                        it = bslot[3 * _CH + rr, pl.ds(tq + o, _LANES)]
                        rrel = bslot[4 * _CH + rr, pl.ds(rq + o, _LANES)]
                        irel = bslot[5 * _CH + rr, pl.ds(rq + o, _LANES)]
                        re = rh * rt + irel * it - rh
                        im = rrel * it - irel * rh - ih
                        s = _vsqrt(re * re + im * im)
                        sv = s if sv is None else sv + s
                    sc[pl.ds(u * _PITCH, _LANES)] = sv
                acc = None
                for i in range(_LANES):
                    col = plsc.load_gather(sc, [col_iota + i])
                    acc = col if acc is None else acc + col
                outv[pl.ds(g * _LANES, _LANES)] = jnp.float32(12.0) - acc
                return inner

            lax.fori_loop(0, _CH // _LANES, row_body, 0)
            pltpu.sync_copy(outv, out_hbm.at[pl.ds(base + off, _CH)])

        issue(0, 0)

        def pipe_body(i, carry):
            c0 = i * 2
            issue(c0 + 1, 1)
            drain(0)
            compute(c0, 0)

            @pl.when(c0 + 2 < nch)
            def _():
                issue(c0 + 2, 0)

            drain(1)
            compute(c0 + 1, 1)
            return carry

        lax.fori_loop(0, nch // 2, pipe_body, 0)

    return k(h, t, r, rw4, iw4, rr_tab, ir_tab)


def kernel(heads, tails, relations, negative_heads, negative_tails,
           negative_relations, re_ent_w, im_ent_w, rel_w):
    b = heads.shape[0]
    rw4, iw4 = _pack_tables(re_ent_w.T, im_ent_w.T)
    rr_tab, ir_tab = _rel_tables(rel_w.reshape(-1, 128))
    h = jnp.concatenate([heads, negative_heads]).astype(jnp.int32)
    t = jnp.concatenate([tails, negative_tails]).astype(jnp.int32)
    r = jnp.concatenate([relations, negative_relations]).astype(jnp.int32)
    out = _sc_score(h, t, r, rw4, iw4, rr_tab, ir_tab)
    return out[:b], out[b:]


# speedup vs baseline: 3.3598x; 1.1366x over previous
import functools

import jax
import jax.numpy as jnp
from jax import lax
from jax.experimental import pallas as pl
from jax.experimental.pallas import tpu as pltpu
from jax.experimental.pallas import tpu_sc as plsc

DIM = 32
EMB_RANGE = 14.0 / 500.0
PI = 3.141592653589793
_PHASE_DIV = EMB_RANGE / PI

_LANES = 16
_CH = 64
_PITCH = _LANES + 1
_SBAND = 1 << 18
_TBR = 4096


def _pack_tables(re_t, im_t):
    def body(a0, a1, a2, a3, b0, b1, b2, b3, oa_ref, ob_ref):
        sa = jnp.concatenate([a0[...], a1[...], a2[...], a3[...]], axis=0)
        sb = jnp.concatenate([b0[...], b1[...], b2[...], b3[...]], axis=0)
        oa_ref[...] = sa.T
        ob_ref[...] = sb.T

    grid = _SBAND // _TBR
    n = re_t.shape[1]
    last_blk = (n + _TBR - 1) // _TBR - 1
    in_specs = []
    for _tbl in range(2):
        for b in range(4):
            in_specs.append(pl.BlockSpec(
                (DIM, _TBR),
                lambda g, b=b: (0, jnp.minimum(g + (_SBAND // _TBR) * b,
                                               last_blk))))
    spec_out = pl.BlockSpec((_TBR, 128), lambda g: (g, 0))
    return pl.pallas_call(
        body,
        grid=(grid,),
        in_specs=in_specs,
        out_specs=[spec_out, spec_out],
        out_shape=[jax.ShapeDtypeStruct((_SBAND, 128), jnp.float32)] * 2,
    )(re_t, re_t, re_t, re_t, im_t, im_t, im_t, im_t)


def _rel_tables(rel_w128):
    def body(rel_ref, rr_ref, ir_ref):
        ph = rel_ref[...] / jnp.float32(_PHASE_DIV)
        rr_ref[...] = jnp.cos(ph)
        ir_ref[...] = jnp.sin(ph)

    return pl.pallas_call(
        body,
        out_shape=[jax.ShapeDtypeStruct(rel_w128.shape, jnp.float32)] * 2,
    )(rel_w128)


def _vsqrt(x):
    x = jnp.maximum(x, jnp.float32(1e-30))
    i = lax.bitcast_convert_type(x, jnp.int32)
    i = jnp.int32(0x5F3759DF) - lax.shift_right_arithmetic(i, jnp.int32(1))
    y = lax.bitcast_convert_type(i, jnp.float32)
    half_x = jnp.float32(0.5) * x
    for _ in range(2):
        y = y * (jnp.float32(1.5) - half_x * y * y)
    return x * y


def _sc_score(h, t, r, rw4, iw4, rr_tab, ir_tab):
    rows = h.shape[0]
    mesh = plsc.VectorSubcoreMesh(core_axis_name="c", subcore_axis_name="s")
    nc, ns = mesh.num_cores, mesh.num_subcores
    nw = nc * ns
    bpw = rows // nw
    nch = bpw // _CH
    assert bpw * nw == rows and nch * _CH == bpw and nch % 2 == 0

    @functools.partial(
        pl.kernel,
        out_type=jax.ShapeDtypeStruct((rows,), jnp.float32),
        mesh=mesh,
        scratch_types=[
            pltpu.VMEM((bpw,), jnp.int32),
            pltpu.VMEM((bpw,), jnp.int32),
            pltpu.VMEM((bpw,), jnp.int32),
            pltpu.VMEM((bpw,), jnp.int32),
            pltpu.VMEM((bpw,), jnp.int32),
            pltpu.VMEM((bpw,), jnp.int32),
            pltpu.VMEM((2, 6 * _CH, 128), jnp.float32),
            pltpu.VMEM((_LANES * _PITCH,), jnp.float32),
            pltpu.VMEM((_CH,), jnp.float32),
            pltpu.SemaphoreType.DMA,
            pltpu.SemaphoreType.DMA,
        ],
        compiler_params=pltpu.CompilerParams(needs_layout_passes=False),
    )
    def k(h_hbm, t_hbm, r_hbm, rew_hbm, imw_hbm, rrt_hbm, irt_hbm, out_hbm,
          hidx, tidx, ridx, hp, tp, rp, buf, sc, outv, sem0, sem1):
        cid = lax.axis_index("c")
        sid = lax.axis_index("s")
        wid = sid * nc + cid
        base = wid * bpw
        pltpu.sync_copy(h_hbm.at[pl.ds(base, bpw)], hidx)
        pltpu.sync_copy(t_hbm.at[pl.ds(base, bpw)], tidx)
        pltpu.sync_copy(r_hbm.at[pl.ds(base, bpw)], ridx)

        def pack_body(j, carry):
            o = j * _LANES
            hp[pl.ds(o, _LANES)] = hidx[pl.ds(o, _LANES)] & (_SBAND - 1)
            tp[pl.ds(o, _LANES)] = tidx[pl.ds(o, _LANES)] & (_SBAND - 1)
            rp[pl.ds(o, _LANES)] = lax.shift_right_logical(
                ridx[pl.ds(o, _LANES)], 2)
            return carry

        lax.fori_loop(0, bpw // _LANES, pack_body, 0)

        row_iota = lax.iota(jnp.int32, _LANES)
        col_iota = row_iota * _PITCH
        sems = (sem0, sem1)

        def issue(cc, slot):
            off = cc * _CH
            bslot = buf.at[slot]
            sem = sems[slot]
            pltpu.async_copy(rew_hbm.at[hp.at[pl.ds(off, _CH)]],
                             bslot.at[pl.ds(0, _CH)], sem)
            pltpu.async_copy(rew_hbm.at[tp.at[pl.ds(off, _CH)]],
                             bslot.at[pl.ds(_CH, _CH)], sem)
            pltpu.async_copy(imw_hbm.at[hp.at[pl.ds(off, _CH)]],
                             bslot.at[pl.ds(2 * _CH, _CH)], sem)
            pltpu.async_copy(imw_hbm.at[tp.at[pl.ds(off, _CH)]],
                             bslot.at[pl.ds(3 * _CH, _CH)], sem)
            pltpu.async_copy(rrt_hbm.at[rp.at[pl.ds(off, _CH)]],
                             bslot.at[pl.ds(4 * _CH, _CH)], sem)
            pltpu.async_copy(irt_hbm.at[rp.at[pl.ds(off, _CH)]],
                             bslot.at[pl.ds(5 * _CH, _CH)], sem)

        def drain(slot):
            pltpu.make_async_copy(
                rew_hbm.at[pl.ds(0, 6 * _CH)], buf.at[slot], sems[slot]).wait()

        def compute(cc, slot):
            off = cc * _CH
            bslot = buf.at[slot]

            def row_body(g, inner):
                goff = off + g * _LANES
                hqv = lax.shift_left(
                    lax.shift_right_logical(hidx[pl.ds(goff, _LANES)], 18), 5)
                tqv = lax.shift_left(
                    lax.shift_right_logical(tidx[pl.ds(goff, _LANES)], 18), 5)
                rqv = lax.shift_left(ridx[pl.ds(goff, _LANES)] & 3, 5)
                for u in range(_LANES):
                    rr = g * _LANES + u
                    hq = hqv[u]
                    tq = tqv[u]
                    rq = rqv[u]
                    sv = None
                    for o in (0, _LANES):
                        rh = bslot[rr, pl.ds(hq + o, _LANES)]
                        rt = bslot[_CH + rr, pl.ds(tq + o, _LANES)]
                        ih = bslot[2 * _CH + rr, pl.ds(hq + o, _LANES)]
                        it = bslot[3 * _CH + rr, pl.ds(tq + o, _LANES)]
                        rrel = bslot[4 * _CH + rr, pl.ds(rq + o, _LANES)]
                        irel = bslot[5 * _CH + rr, pl.ds(rq + o, _LANES)]
                        re = rh * rt + irel * it - rh
                        im = rrel * it - irel * rh - ih
                        s = _vsqrt(re * re + im * im)
                        sv = s if sv is None else sv + s
                    sc[pl.ds(u * _PITCH, _LANES)] = sv
                acc = None
                for i in range(_LANES):
                    col = plsc.load_gather(sc, [col_iota + i])
                    acc = col if acc is None else acc + col
                outv[pl.ds(g * _LANES, _LANES)] = jnp.float32(12.0) - acc
                return inner

            lax.fori_loop(0, _CH // _LANES, row_body, 0)
            pltpu.sync_copy(outv, out_hbm.at[pl.ds(base + off, _CH)])

        issue(0, 0)

        def pipe_body(i, carry):
            c0 = i * 2
            issue(c0 + 1, 1)
            drain(0)
            compute(c0, 0)

            @pl.when(c0 + 2 < nch)
            def _():
                issue(c0 + 2, 0)

            drain(1)
            compute(c0 + 1, 1)
            return carry

        lax.fori_loop(0, nch // 2, pipe_body, 0)

    return k(h, t, r, rw4, iw4, rr_tab, ir_tab)


def kernel(heads, tails, relations, negative_heads, negative_tails,
           negative_relations, re_ent_w, im_ent_w, rel_w):
    b = heads.shape[0]
    rw4, iw4 = _pack_tables(re_ent_w.T, im_ent_w.T)
    rr_tab, ir_tab = _rel_tables(rel_w.reshape(-1, 128))
    h = jnp.concatenate([heads, negative_heads]).astype(jnp.int32)
    t = jnp.concatenate([tails, negative_tails]).astype(jnp.int32)
    r = jnp.concatenate([relations, negative_relations]).astype(jnp.int32)
    out = _sc_score(h, t, r, rw4, iw4, rr_tab, ir_tab)
    return out[:b], out[b:]


# speedup vs baseline: 3.4437x; 1.0250x over previous
import functools

import jax
import jax.numpy as jnp
from jax import lax
from jax.experimental import pallas as pl
from jax.experimental.pallas import tpu as pltpu
from jax.experimental.pallas import tpu_sc as plsc

DIM = 32
EMB_RANGE = 14.0 / 500.0
PI = 3.141592653589793
_PHASE_DIV = EMB_RANGE / PI

_LANES = 16
_CH = 64
_PITCH = _LANES + 1
_SBAND = 1 << 18
_TBR = 8192


def _pack_tables(re_t, im_t):
    def body(a0, a1, a2, a3, b0, b1, b2, b3, oa_ref, ob_ref):
        sa = jnp.concatenate([a0[...], a1[...], a2[...], a3[...]], axis=0)
        sb = jnp.concatenate([b0[...], b1[...], b2[...], b3[...]], axis=0)
        oa_ref[...] = sa.T
        ob_ref[...] = sb.T

    grid = _SBAND // _TBR
    n = re_t.shape[1]
    last_blk = (n + _TBR - 1) // _TBR - 1
    in_specs = []
    for _tbl in range(2):
        for b in range(4):
            in_specs.append(pl.BlockSpec(
                (DIM, _TBR),
                lambda g, b=b: (0, jnp.minimum(g + (_SBAND // _TBR) * b,
                                               last_blk))))
    spec_out = pl.BlockSpec((_TBR, 128), lambda g: (g, 0))
    return pl.pallas_call(
        body,
        grid=(grid,),
        in_specs=in_specs,
        out_specs=[spec_out, spec_out],
        out_shape=[jax.ShapeDtypeStruct((_SBAND, 128), jnp.float32)] * 2,
    )(re_t, re_t, re_t, re_t, im_t, im_t, im_t, im_t)


def _rel_tables(rel_w128):
    def body(rel_ref, rr_ref, ir_ref):
        ph = rel_ref[...] / jnp.float32(_PHASE_DIV)
        rr_ref[...] = jnp.cos(ph)
        ir_ref[...] = jnp.sin(ph)

    return pl.pallas_call(
        body,
        out_shape=[jax.ShapeDtypeStruct(rel_w128.shape, jnp.float32)] * 2,
    )(rel_w128)


def _vsqrt(x):
    x = jnp.maximum(x, jnp.float32(1e-30))
    i = lax.bitcast_convert_type(x, jnp.int32)
    i = jnp.int32(0x5F3759DF) - lax.shift_right_arithmetic(i, jnp.int32(1))
    y = lax.bitcast_convert_type(i, jnp.float32)
    half_x = jnp.float32(0.5) * x
    for _ in range(2):
        y = y * (jnp.float32(1.5) - half_x * y * y)
    return x * y


def _sc_score(h, t, r, rw4, iw4, rr_tab, ir_tab):
    rows = h.shape[0]
    mesh = plsc.VectorSubcoreMesh(core_axis_name="c", subcore_axis_name="s")
    nc, ns = mesh.num_cores, mesh.num_subcores
    nw = nc * ns
    bpw = rows // nw
    nch = bpw // _CH
    assert bpw * nw == rows and nch * _CH == bpw and nch % 2 == 0

    @functools.partial(
        pl.kernel,
        out_type=jax.ShapeDtypeStruct((rows,), jnp.float32),
        mesh=mesh,
        scratch_types=[
            pltpu.VMEM((bpw,), jnp.int32),
            pltpu.VMEM((bpw,), jnp.int32),
            pltpu.VMEM((bpw,), jnp.int32),
            pltpu.VMEM((bpw,), jnp.int32),
            pltpu.VMEM((bpw,), jnp.int32),
            pltpu.VMEM((bpw,), jnp.int32),
            pltpu.VMEM((2, 6 * _CH, 128), jnp.float32),
            pltpu.VMEM((_LANES * _PITCH,), jnp.float32),
            pltpu.VMEM((_CH,), jnp.float32),
            pltpu.SemaphoreType.DMA,
            pltpu.SemaphoreType.DMA,
        ],
        compiler_params=pltpu.CompilerParams(needs_layout_passes=False),
    )
    def k(h_hbm, t_hbm, r_hbm, rew_hbm, imw_hbm, rrt_hbm, irt_hbm, out_hbm,
          hidx, tidx, ridx, hp, tp, rp, buf, sc, outv, sem0, sem1):
        cid = lax.axis_index("c")
        sid = lax.axis_index("s")
        wid = sid * nc + cid
        base = wid * bpw
        pltpu.sync_copy(h_hbm.at[pl.ds(base, bpw)], hidx)
        pltpu.sync_copy(t_hbm.at[pl.ds(base, bpw)], tidx)
        pltpu.sync_copy(r_hbm.at[pl.ds(base, bpw)], ridx)

        def pack_body(j, carry):
            o = j * _LANES
            hp[pl.ds(o, _LANES)] = hidx[pl.ds(o, _LANES)] & (_SBAND - 1)
            tp[pl.ds(o, _LANES)] = tidx[pl.ds(o, _LANES)] & (_SBAND - 1)
            rp[pl.ds(o, _LANES)] = lax.shift_right_logical(
                ridx[pl.ds(o, _LANES)], 2)
            return carry

        lax.fori_loop(0, bpw // _LANES, pack_body, 0)

        row_iota = lax.iota(jnp.int32, _LANES)
        col_iota = row_iota * _PITCH
        sems = (sem0, sem1)

        def issue(cc, slot):
            off = cc * _CH
            bslot = buf.at[slot]
            sem = sems[slot]
            pltpu.async_copy(rew_hbm.at[hp.at[pl.ds(off, _CH)]],
                             bslot.at[pl.ds(0, _CH)], sem)
            pltpu.async_copy(rew_hbm.at[tp.at[pl.ds(off, _CH)]],
                             bslot.at[pl.ds(_CH, _CH)], sem)
            pltpu.async_copy(imw_hbm.at[hp.at[pl.ds(off, _CH)]],
                             bslot.at[pl.ds(2 * _CH, _CH)], sem)
            pltpu.async_copy(imw_hbm.at[tp.at[pl.ds(off, _CH)]],
                             bslot.at[pl.ds(3 * _CH, _CH)], sem)
            pltpu.async_copy(rrt_hbm.at[rp.at[pl.ds(off, _CH)]],
                             bslot.at[pl.ds(4 * _CH, _CH)], sem)
            pltpu.async_copy(irt_hbm.at[rp.at[pl.ds(off, _CH)]],
                             bslot.at[pl.ds(5 * _CH, _CH)], sem)

        def drain(slot):
            pltpu.make_async_copy(
                rew_hbm.at[pl.ds(0, 6 * _CH)], buf.at[slot], sems[slot]).wait()

        def compute(cc, slot):
            off = cc * _CH
            bslot = buf.at[slot]

            def row_body(g, inner):
                goff = off + g * _LANES
                hqv = lax.shift_left(
                    lax.shift_right_logical(hidx[pl.ds(goff, _LANES)], 18), 5)
                tqv = lax.shift_left(
                    lax.shift_right_logical(tidx[pl.ds(goff, _LANES)], 18), 5)
                rqv = lax.shift_left(ridx[pl.ds(goff, _LANES)] & 3, 5)
                for u in range(_LANES):
                    rr = g * _LANES + u
                    hq = hqv[u]
                    tq = tqv[u]
                    rq = rqv[u]
                    sv = None
                    for o in (0, _LANES):
                        rh = bslot[rr, pl.ds(hq + o, _LANES)]
                        rt = bslot[_CH + rr, pl.ds(tq + o, _LANES)]
                        ih = bslot[2 * _CH + rr, pl.ds(hq + o, _LANES)]
                        it = bslot[3 * _CH + rr, pl.ds(tq + o, _LANES)]
                        rrel = bslot[4 * _CH + rr, pl.ds(rq + o, _LANES)]
                        irel = bslot[5 * _CH + rr, pl.ds(rq + o, _LANES)]
                        re = rh * rt + irel * it - rh
                        im = rrel * it - irel * rh - ih
                        s = _vsqrt(re * re + im * im)
                        sv = s if sv is None else sv + s
                    sc[pl.ds(u * _PITCH, _LANES)] = sv
                acc = None
                for i in range(_LANES):
                    col = plsc.load_gather(sc, [col_iota + i])
                    acc = col if acc is None else acc + col
                outv[pl.ds(g * _LANES, _LANES)] = jnp.float32(12.0) - acc
                return inner

            lax.fori_loop(0, _CH // _LANES, row_body, 0)
            pltpu.sync_copy(outv, out_hbm.at[pl.ds(base + off, _CH)])

        issue(0, 0)

        def pipe_body(i, carry):
            c0 = i * 2
            issue(c0 + 1, 1)
            drain(0)
            compute(c0, 0)

            @pl.when(c0 + 2 < nch)
            def _():
                issue(c0 + 2, 0)

            drain(1)
            compute(c0 + 1, 1)
            return carry

        lax.fori_loop(0, nch // 2, pipe_body, 0)

    return k(h, t, r, rw4, iw4, rr_tab, ir_tab)


def kernel(heads, tails, relations, negative_heads, negative_tails,
           negative_relations, re_ent_w, im_ent_w, rel_w):
    b = heads.shape[0]
    rw4, iw4 = _pack_tables(re_ent_w.T, im_ent_w.T)
    rr_tab, ir_tab = _rel_tables(rel_w.reshape(-1, 128))
    h = jnp.concatenate([heads, negative_heads]).astype(jnp.int32)
    t = jnp.concatenate([tails, negative_tails]).astype(jnp.int32)
    r = jnp.concatenate([relations, negative_relations]).astype(jnp.int32)
    out = _sc_score(h, t, r, rw4, iw4, rr_tab, ir_tab)
    return out[:b], out[b:]
